# Initial kernel scaffold; baseline (speedup 1.0000x reference)
import functools

import jax
import jax.numpy as jnp
from jax import lax
from jax.experimental import pallas as pl
from jax.experimental.pallas import tpu as pltpu
from jax.experimental.pallas import tpu_sc as plsc

N = 10000
NP = 10240
E = 160000
EP = 163840
D = 256
HD = 128
CN = 64
NTILES = 16
CHUNK = 128
CPT = EP // CHUNK // NTILES
RPT = NP // NTILES
RING = 4
GROUPS = CPT // RING
BN_BLK = 512
GRID = NP // BN_BLK


def _make_sc_agg(compute_deg):
    mesh = plsc.VectorSubcoreMesh(core_axis_name="c", subcore_axis_name="s")
    outs = [jax.ShapeDtypeStruct((2, NP, HD), jnp.float32)]
    if compute_deg:
        outs.append(jax.ShapeDtypeStruct((NTILES, CPT, CHUNK), jnp.float32))
    scratch = [
        pltpu.VMEM((CPT, CHUNK), jnp.int32),
        pltpu.VMEM((CPT, CHUNK), jnp.int32),
        pltpu.VMEM((RING, CHUNK, HD), jnp.float32),
    ]
    if compute_deg:
        scratch.append(pltpu.VMEM((CPT, CHUNK), jnp.float32))
    scratch.append(pltpu.VMEM_SHARED((NP, HD), jnp.float32))
    scratch += [pltpu.SemaphoreType.DMA] * (2 * RING)

    def body(*refs):
        if compute_deg:
            (h_hbm, src_hbm, dst_hbm, zeros_hbm, out_hbm, deg_out,
             srcbuf, dstbuf, rows, degbuf, agg, *sems) = refs
        else:
            (h_hbm, src_hbm, dst_hbm, zeros_hbm, out_hbm,
             srcbuf, dstbuf, rows, agg, *sems) = refs
            deg_out = degbuf = None
        gsem = sems[:RING]
        ssem = sems[RING:]
        cid = lax.axis_index("c")
        sid = lax.axis_index("s")

        pltpu.sync_copy(src_hbm.at[cid, pl.ds(sid * CPT, CPT)], srcbuf)
        pltpu.sync_copy(dst_hbm.at[pl.ds(sid * CPT, CPT)], dstbuf)
        pltpu.sync_copy(zeros_hbm, rows.at[0])
        for z in range(RPT // CHUNK):
            pltpu.sync_copy(
                rows.at[0], agg.at[pl.ds(sid * RPT + z * CHUNK, CHUNK)])
        plsc.subcore_barrier()

        def gather_start(j, b):
            pltpu.async_copy(h_hbm.at[srcbuf.at[j]], rows.at[b], gsem[b])

        def gather_wait(j, b):
            pltpu.make_async_copy(
                h_hbm.at[srcbuf.at[j]], rows.at[b], gsem[b]).wait()

        def scat_start(j, b):
            pltpu.async_copy(
                rows.at[b], agg.at[dstbuf.at[j]], ssem[b], add=True)

        def scat_wait(j, b):
            pltpu.make_async_copy(
                rows.at[b], agg.at[dstbuf.at[j]], ssem[b]).wait()

        for b in range(RING):
            gather_start(b, b)

        def grp(g, carry):
            for b in range(RING):
                j = g * RING + b
                gather_wait(j, b)
                scat_start(j, b)
                scat_wait(j, b)

                @pl.when(g < GROUPS - 1)
                def _():
                    gather_start(j + RING, b)
            return carry

        lax.fori_loop(0, GROUPS, grp, 0)

        if compute_deg:
            @pl.when(cid == 0)
            def _():
                pltpu.sync_copy(zeros_hbm.at[pl.ds(0, CPT)], degbuf)
                ones16 = jnp.full((16,), 1.0, jnp.float32)

                def degloop(i, carry):
                    r = i // 8
                    c = (i % 8) * 16
                    d16 = dstbuf[r, pl.ds(c, 16)]
                    plsc.addupdate_scatter(
                        degbuf,
                        [lax.shift_right_logical(d16, 7),
                         lax.bitwise_and(d16, 127)],
                        ones16)
                    return carry

                lax.fori_loop(0, CPT * 8, degloop, 0)
                pltpu.sync_copy(degbuf, deg_out.at[sid])

        plsc.subcore_barrier()
        pltpu.sync_copy(agg.at[pl.ds(sid * RPT, RPT)],
                        out_hbm.at[cid, pl.ds(sid * RPT, RPT)])

    return pl.kernel(body, out_type=tuple(outs) if compute_deg else outs[0],
                     mesh=mesh, scratch_types=scratch)


def _tc_invdeg(degp):
    def body(d_ref, o_ref):
        dsum = jnp.sum(d_ref[...], axis=0)
        o_ref[...] = 1.0 / jnp.maximum(dsum, 1.0)

    return pl.pallas_call(
        body, out_shape=jax.ShapeDtypeStruct((CPT, CHUNK), jnp.float32))(degp)


def _tc_layer(aggh, hh, invd, Wl, bl, Wr):
    def body(agg_ref, h_ref, invd_ref, wl_ref, bl_ref, wr_ref, y_ref, p_ref):
        g = pl.program_id(0)
        iv = invd_ref[...].reshape(BN_BLK, 1)
        wl = wl_ref[...]
        wr = wr_ref[...]
        y = (jnp.dot(agg_ref[0] * iv, wl[:HD],
                     preferred_element_type=jnp.float32)
             + jnp.dot(agg_ref[1] * iv, wl[HD:],
                       preferred_element_type=jnp.float32)
             + jnp.dot(h_ref[0], wr[:HD], preferred_element_type=jnp.float32)
             + jnp.dot(h_ref[1], wr[HD:], preferred_element_type=jnp.float32)
             + bl_ref[...])
        y_ref[...] = y
        rows = g * BN_BLK + lax.broadcasted_iota(jnp.int32, (BN_BLK, 1), 0)
        ym = jnp.where(rows < N, y, 0.0)
        p = jnp.zeros((8, D), jnp.float32)
        p = p.at[0].set(jnp.sum(ym, axis=0)).at[1].set(jnp.sum(ym * ym, axis=0))
        p_ref[...] = p.reshape(1, 8, D)

    return pl.pallas_call(
        body,
        grid=(GRID,),
        in_specs=[
            pl.BlockSpec((2, BN_BLK, HD), lambda g: (0, g, 0)),
            pl.BlockSpec((2, BN_BLK, HD), lambda g: (0, g, 0)),
            pl.BlockSpec((BN_BLK // CHUNK, CHUNK), lambda g: (g, 0)),
            pl.BlockSpec((D, D), lambda g: (0, 0)),
            pl.BlockSpec((1, D), lambda g: (0, 0)),
            pl.BlockSpec((D, D), lambda g: (0, 0)),
        ],
        out_specs=[
            pl.BlockSpec((BN_BLK, D), lambda g: (g, 0)),
            pl.BlockSpec((1, 8, D), lambda g: (g, 0, 0)),
        ],
        out_shape=[
            jax.ShapeDtypeStruct((NP, D), jnp.float32),
            jax.ShapeDtypeStruct((GRID, 8, D), jnp.float32),
        ],
    )(aggh, hh, invd, Wl, bl, Wr)


def _tc_norm(y, p, gamma, beta):
    def body(y_ref, p_ref, g_ref, b_ref, o_ref):
        gidx = pl.program_id(0)
        ps = p_ref[...]
        mu = jnp.sum(ps[:, 0, :], axis=0) / N
        var = jnp.sum(ps[:, 1, :], axis=0) / N - mu * mu
        sc = g_ref[0] * lax.rsqrt(var + 1e-5)
        t = b_ref[0] - mu * sc
        h = jnp.maximum(y_ref[...] * sc + t, 0.0)
        rows = gidx * BN_BLK + lax.broadcasted_iota(jnp.int32, (BN_BLK, 1), 0)
        h = jnp.where(rows < N, h, 0.0)
        o_ref[0] = h[:, :HD]
        o_ref[1] = h[:, HD:]

    return pl.pallas_call(
        body,
        grid=(GRID,),
        in_specs=[
            pl.BlockSpec((BN_BLK, D), lambda g: (g, 0)),
            pl.BlockSpec((GRID, 8, D), lambda g: (0, 0, 0)),
            pl.BlockSpec((1, D), lambda g: (0, 0)),
            pl.BlockSpec((1, D), lambda g: (0, 0)),
        ],
        out_specs=pl.BlockSpec((2, BN_BLK, HD), lambda g: (0, g, 0)),
        out_shape=jax.ShapeDtypeStruct((2, NP, HD), jnp.float32),
    )(y, p, gamma, beta)


def _tc_head(h1, h2, h3, W1, b1, W2, b2):
    def body(h1_ref, h2_ref, h3_ref, w1_ref, b1_ref, w2_ref, b2_ref, o_ref):
        w1 = w1_ref[...]
        acc = (jnp.dot(h1_ref[0], w1[0], preferred_element_type=jnp.float32)
               + jnp.dot(h1_ref[1], w1[1], preferred_element_type=jnp.float32)
               + jnp.dot(h2_ref[0], w1[2], preferred_element_type=jnp.float32)
               + jnp.dot(h2_ref[1], w1[3], preferred_element_type=jnp.float32)
               + jnp.dot(h3_ref[0], w1[4], preferred_element_type=jnp.float32)
               + jnp.dot(h3_ref[1], w1[5], preferred_element_type=jnp.float32)
               + b1_ref[...])
        u = jnp.maximum(acc, 0.0)
        v = jnp.dot(u, w2_ref[...], preferred_element_type=jnp.float32) \
            + b2_ref[...]
        m = jnp.max(v, axis=1, keepdims=True)
        lse = jnp.log(jnp.sum(jnp.exp(v - m), axis=1, keepdims=True)) + m
        o_ref[...] = v - lse

    hspec = pl.BlockSpec((2, BN_BLK, HD), lambda g: (0, g, 0))
    return pl.pallas_call(
        body,
        grid=(GRID,),
        in_specs=[
            hspec, hspec, hspec,
            pl.BlockSpec((6, HD, D), lambda g: (0, 0, 0)),
            pl.BlockSpec((1, D), lambda g: (0, 0)),
            pl.BlockSpec((D, CN), lambda g: (0, 0)),
            pl.BlockSpec((1, CN), lambda g: (0, 0)),
        ],
        out_specs=pl.BlockSpec((BN_BLK, CN), lambda g: (g, 0)),
        out_shape=jax.ShapeDtypeStruct((NP, CN), jnp.float32),
    )(h1, h2, h3, W1, b1, W2, b2)


def kernel(x, edge_index, W_l_0, b_l_0, W_r_0, bn_g_0, bn_b_0,
           W_l_1, b_l_1, W_r_1, bn_g_1, bn_b_1,
           W_l_2, b_l_2, W_r_2, bn_g_2, bn_b_2,
           lin1_W, lin1_b, lin2_W, lin2_b):
    f32 = jnp.float32
    pad = EP - E
    srcp = jnp.concatenate([edge_index[0], jnp.full((pad,), N, jnp.int32)])
    dstp = jnp.concatenate([edge_index[1], jnp.full((pad,), N, jnp.int32)])
    src2 = jnp.stack([srcp, srcp + NP]).reshape(2, EP // CHUNK, CHUNK)
    dst2 = dstp.reshape(EP // CHUNK, CHUNK)
    zeros = jnp.zeros((CHUNK, HD), f32)
    hh = jnp.pad(x, ((0, NP - N), (0, 0))).reshape(NP, 2, HD).transpose(1, 0, 2)

    sc0 = _make_sc_agg(True)
    scn = _make_sc_agg(False)
    layer_params = [
        (W_l_0, b_l_0, W_r_0, bn_g_0, bn_b_0),
        (W_l_1, b_l_1, W_r_1, bn_g_1, bn_b_1),
        (W_l_2, b_l_2, W_r_2, bn_g_2, bn_b_2),
    ]
    invd = None
    feats = []
    for i, (Wl, bl, Wr, g, b) in enumerate(layer_params):
        hflat = hh.reshape(2 * NP, HD)
        if i == 0:
            aggh, degp = sc0(hflat, src2, dst2, zeros)
            invd = _tc_invdeg(degp)
        else:
            aggh = scn(hflat, src2, dst2, zeros)
        y, p = _tc_layer(aggh, hh, invd, Wl, bl.reshape(1, D), Wr)
        hh = _tc_norm(y, p, g.reshape(1, D), b.reshape(1, D))
        feats.append(hh)

    out = _tc_head(feats[0], feats[1], feats[2],
                   lin1_W.reshape(6, HD, D), lin1_b.reshape(1, D),
                   lin2_W, lin2_b.reshape(1, CN))
    return out[:N]


# speedup vs baseline: 2.5699x; 2.5699x over previous
import jax
import jax.numpy as jnp
from jax import lax
from jax.experimental import pallas as pl
from jax.experimental.pallas import tpu as pltpu
from jax.experimental.pallas import tpu_sc as plsc

N = 10000
NP = 10240
E = 160000
EP = 163840
D = 256
QD = 64
CN = 64
NTILES = 16
CHUNK = 128
CPT = EP // CHUNK // NTILES
RPT = NP // NTILES
RING = 4
GROUPS = CPT // RING
DEGW = 16
BN_BLK = 512
GRID = NP // BN_BLK


def _make_sc_agg(compute_deg):
    mesh = plsc.VectorSubcoreMesh(core_axis_name="c", subcore_axis_name="s",
                                  num_cores=2, num_subcores=NTILES)
    outs = [jax.ShapeDtypeStruct((4, NP, QD), jnp.float32)]
    if compute_deg:
        outs.append(jax.ShapeDtypeStruct((NP, DEGW), jnp.float32))
    scratch = [
        pltpu.VMEM((CPT, CHUNK), jnp.int32),
        pltpu.VMEM((CPT, CHUNK), jnp.int32),
        pltpu.VMEM((RING, CHUNK, QD), jnp.float32),
        pltpu.VMEM_SHARED((NP, QD), jnp.float32),
    ]
    if compute_deg:
        scratch += [
            pltpu.VMEM((CHUNK, DEGW), jnp.float32),
            pltpu.VMEM_SHARED((NP, DEGW), jnp.float32),
        ]
    scratch += [pltpu.SemaphoreType.DMA] * (2 * RING + 1)

    def body(*refs):
        if compute_deg:
            (h_hbm, src_hbm, dst_hbm, zeros_hbm, ones_hbm, zerosd_hbm,
             out_hbm, deg_out,
             srcbuf, dstbuf, rows, agg, ones, degacc, *sems) = refs
        else:
            (h_hbm, src_hbm, dst_hbm, zeros_hbm, out_hbm,
             srcbuf, dstbuf, rows, agg, *sems) = refs
            deg_out = ones = degacc = ones_hbm = None
        gsem = sems[:RING]
        ssem = sems[RING:2 * RING]
        dsem = sems[2 * RING]
        cid = lax.axis_index("c")
        sid = lax.axis_index("s")

        pltpu.sync_copy(src_hbm.at[cid, pl.ds(sid * CPT, CPT)], srcbuf)
        pltpu.sync_copy(dst_hbm.at[pl.ds(sid * CPT, CPT)], dstbuf)
        if compute_deg:
            pltpu.sync_copy(ones_hbm, ones)
            for z in range(RPT // CHUNK):
                pltpu.sync_copy(
                    zerosd_hbm,
                    degacc.at[pl.ds(sid * RPT + z * CHUNK, CHUNK)])

        def gather_start(j, b):
            pltpu.async_copy(h_hbm.at[srcbuf.at[j]], rows.at[b], gsem[b])

        def gather_wait(j, b):
            pltpu.make_async_copy(
                h_hbm.at[srcbuf.at[j]], rows.at[b], gsem[b]).wait()

        def scat_start(j, b):
            pltpu.async_copy(
                rows.at[b], agg.at[dstbuf.at[j]], ssem[b], add=True)

        def scat_wait(j, b):
            pltpu.make_async_copy(
                rows.at[b], agg.at[dstbuf.at[j]], ssem[b]).wait()

        for p in range(2):
            if p == 1:
                bump = jnp.full((16,), 2 * NP, jnp.int32)

                def bumploop(i, carry):
                    r = i // 8
                    c = (i % 8) * 16
                    srcbuf[r, pl.ds(c, 16)] = srcbuf[r, pl.ds(c, 16)] + bump
                    return carry

                lax.fori_loop(0, CPT * 8, bumploop, 0)

            pltpu.sync_copy(zeros_hbm, rows.at[0])
            for z in range(RPT // CHUNK):
                pltpu.sync_copy(
                    rows.at[0], agg.at[pl.ds(sid * RPT + z * CHUNK, CHUNK)])
            plsc.subcore_barrier()

            for b in range(RING):
                gather_start(b, b)

            def grp(g, carry):
                for b in range(RING):
                    j = g * RING + b
                    gather_wait(j, b)
                    scat_start(j, b)
                    scat_wait(j, b)

                    @pl.when(g < GROUPS - 1)
                    def _():
                        gather_start(j + RING, b)
                return carry

            lax.fori_loop(0, GROUPS, grp, 0)

            if compute_deg and p == 0:
                @pl.when(cid == 0)
                def _():
                    def degfire(j, carry):
                        pltpu.async_copy(
                            ones, degacc.at[dstbuf.at[j]], dsem, add=True)
                        return carry

                    lax.fori_loop(0, CPT, degfire, 0)

                    def degdrain(j, carry):
                        pltpu.make_async_copy(
                            ones, degacc.at[dstbuf.at[0]], dsem).wait()
                        return carry

                    lax.fori_loop(0, CPT, degdrain, 0)

            plsc.subcore_barrier()
            pltpu.sync_copy(
                agg.at[pl.ds(sid * RPT, RPT)],
                out_hbm.at[cid + 2 * p, pl.ds(sid * RPT, RPT)])
            if compute_deg and p == 0:
                @pl.when(cid == 0)
                def _():
                    pltpu.sync_copy(degacc.at[pl.ds(sid * RPT, RPT)],
                                    deg_out.at[pl.ds(sid * RPT, RPT)])

    return pl.kernel(body, out_type=tuple(outs) if compute_deg else outs[0],
                     mesh=mesh, scratch_types=scratch,
                     compiler_params=pltpu.CompilerParams(
                         use_tc_tiling_on_sc=False))


def _tc_invdeg(degp):
    def body(d_ref, o_ref):
        deg = d_ref[...][:, 0:1]
        o_ref[...] = 1.0 / jnp.maximum(deg, 1.0)

    return pl.pallas_call(
        body, out_shape=jax.ShapeDtypeStruct((NP, 1), jnp.float32))(degp)


def _tc_layer(aggh, hh, invd, Wl, bl, Wr):
    def body(agg_ref, h_ref, invd_ref, wl_ref, bl_ref, wr_ref, y_ref, p_ref):
        g = pl.program_id(0)
        iv = invd_ref[...]
        wl = wl_ref[...]
        wr = wr_ref[...]
        y = bl_ref[...]
        for k in range(4):
            y = y + jnp.dot(agg_ref[k] * iv, wl[k * QD:(k + 1) * QD],
                            preferred_element_type=jnp.float32)
            y = y + jnp.dot(h_ref[k], wr[k * QD:(k + 1) * QD],
                            preferred_element_type=jnp.float32)
        y_ref[...] = y
        rows = g * BN_BLK + lax.broadcasted_iota(jnp.int32, (BN_BLK, 1), 0)
        ym = jnp.where(rows < N, y, 0.0)
        p = jnp.concatenate(
            [jnp.sum(ym, axis=0)[None], jnp.sum(ym * ym, axis=0)[None],
             jnp.zeros((6, D), jnp.float32)], axis=0)
        p_ref[...] = p.reshape(1, 8, D)

    return pl.pallas_call(
        body,
        grid=(GRID,),
        in_specs=[
            pl.BlockSpec((4, BN_BLK, QD), lambda g: (0, g, 0)),
            pl.BlockSpec((4, BN_BLK, QD), lambda g: (0, g, 0)),
            pl.BlockSpec((BN_BLK, 1), lambda g: (g, 0)),
            pl.BlockSpec((D, D), lambda g: (0, 0)),
            pl.BlockSpec((1, D), lambda g: (0, 0)),
            pl.BlockSpec((D, D), lambda g: (0, 0)),
        ],
        out_specs=[
            pl.BlockSpec((BN_BLK, D), lambda g: (g, 0)),
            pl.BlockSpec((1, 8, D), lambda g: (g, 0, 0)),
        ],
        out_shape=[
            jax.ShapeDtypeStruct((NP, D), jnp.float32),
            jax.ShapeDtypeStruct((GRID, 8, D), jnp.float32),
        ],
    )(aggh, hh, invd, Wl, bl, Wr)


def _tc_norm(y, p, gamma, beta):
    def body(y_ref, p_ref, g_ref, b_ref, o_ref):
        gidx = pl.program_id(0)
        ps = p_ref[...]
        mu = jnp.sum(ps[:, 0, :], axis=0) / N
        var = jnp.sum(ps[:, 1, :], axis=0) / N - mu * mu
        sc = g_ref[0] * lax.rsqrt(var + 1e-5)
        t = b_ref[0] - mu * sc
        h = jnp.maximum(y_ref[...] * sc + t, 0.0)
        rows = gidx * BN_BLK + lax.broadcasted_iota(jnp.int32, (BN_BLK, 1), 0)
        h = jnp.where(rows < N, h, 0.0)
        for k in range(4):
            o_ref[k] = h[:, k * QD:(k + 1) * QD]

    return pl.pallas_call(
        body,
        grid=(GRID,),
        in_specs=[
            pl.BlockSpec((BN_BLK, D), lambda g: (g, 0)),
            pl.BlockSpec((GRID, 8, D), lambda g: (0, 0, 0)),
            pl.BlockSpec((1, D), lambda g: (0, 0)),
            pl.BlockSpec((1, D), lambda g: (0, 0)),
        ],
        out_specs=pl.BlockSpec((4, BN_BLK, QD), lambda g: (0, g, 0)),
        out_shape=jax.ShapeDtypeStruct((4, NP, QD), jnp.float32),
    )(y, p, gamma, beta)


def _tc_head(h1, h2, h3, W1, b1, W2, b2):
    def body(h1_ref, h2_ref, h3_ref, w1_ref, b1_ref, w2_ref, b2_ref, o_ref):
        w1 = w1_ref[...]
        acc = b1_ref[...]
        for li, href in enumerate((h1_ref, h2_ref, h3_ref)):
            for k in range(4):
                acc = acc + jnp.dot(href[k], w1[4 * li + k],
                                    preferred_element_type=jnp.float32)
        u = jnp.maximum(acc, 0.0)
        v = jnp.dot(u, w2_ref[...], preferred_element_type=jnp.float32) \
            + b2_ref[...]
        m = jnp.max(v, axis=1, keepdims=True)
        lse = jnp.log(jnp.sum(jnp.exp(v - m), axis=1, keepdims=True)) + m
        o_ref[...] = v - lse

    hspec = pl.BlockSpec((4, BN_BLK, QD), lambda g: (0, g, 0))
    return pl.pallas_call(
        body,
        grid=(GRID,),
        in_specs=[
            hspec, hspec, hspec,
            pl.BlockSpec((12, QD, D), lambda g: (0, 0, 0)),
            pl.BlockSpec((1, D), lambda g: (0, 0)),
            pl.BlockSpec((D, CN), lambda g: (0, 0)),
            pl.BlockSpec((1, CN), lambda g: (0, 0)),
        ],
        out_specs=pl.BlockSpec((BN_BLK, CN), lambda g: (g, 0)),
        out_shape=jax.ShapeDtypeStruct((NP, CN), jnp.float32),
    )(h1, h2, h3, W1, b1, W2, b2)


def kernel(x, edge_index, W_l_0, b_l_0, W_r_0, bn_g_0, bn_b_0,
           W_l_1, b_l_1, W_r_1, bn_g_1, bn_b_1,
           W_l_2, b_l_2, W_r_2, bn_g_2, bn_b_2,
           lin1_W, lin1_b, lin2_W, lin2_b):
    f32 = jnp.float32
    pad = EP - E
    srcp = jnp.concatenate([edge_index[0], jnp.full((pad,), N, jnp.int32)])
    dstp = jnp.concatenate([edge_index[1], jnp.full((pad,), N, jnp.int32)])
    src2 = jnp.stack([srcp, srcp + NP]).reshape(2, EP // CHUNK, CHUNK)
    dst2 = dstp.reshape(EP // CHUNK, CHUNK)
    zeros = jnp.zeros((CHUNK, QD), f32)
    ones = jnp.ones((CHUNK, DEGW), f32)
    zerosd = jnp.zeros((CHUNK, DEGW), f32)
    hh = jnp.pad(x, ((0, NP - N), (0, 0))).reshape(NP, 4, QD).transpose(1, 0, 2)

    sc0 = _make_sc_agg(True)
    scn = _make_sc_agg(False)
    layer_params = [
        (W_l_0, b_l_0, W_r_0, bn_g_0, bn_b_0),
        (W_l_1, b_l_1, W_r_1, bn_g_1, bn_b_1),
        (W_l_2, b_l_2, W_r_2, bn_g_2, bn_b_2),
    ]
    invd = None
    feats = []
    for i, (Wl, bl, Wr, g, b) in enumerate(layer_params):
        hflat = hh.reshape(4 * NP, QD)
        if i == 0:
            aggh, degp = sc0(hflat, src2, dst2, zeros, ones, zerosd)
            invd = _tc_invdeg(degp)
        else:
            aggh = scn(hflat, src2, dst2, zeros)
        y, p = _tc_layer(aggh, hh, invd, Wl, bl.reshape(1, D), Wr)
        hh = _tc_norm(y, p, g.reshape(1, D), b.reshape(1, D))
        feats.append(hh)

    out = _tc_head(feats[0], feats[1], feats[2],
                   lin1_W.reshape(12, QD, D), lin1_b.reshape(1, D),
                   lin2_W, lin2_b.reshape(1, CN))
    return out[:N]


# speedup vs baseline: 2.5705x; 1.0002x over previous
import jax
import jax.numpy as jnp
from jax import lax
from jax.experimental import pallas as pl
from jax.experimental.pallas import tpu as pltpu
from jax.experimental.pallas import tpu_sc as plsc

N = 10000
NP = 10240
E = 160000
EP = 163840
D = 256
QD = 64
CN = 64
NTILES = 16
CHUNK = 128
CPT = EP // CHUNK // NTILES
RPT = NP // NTILES

DEGW = 16
BN_BLK = 512
GRID = NP // BN_BLK


def _make_sc_agg(compute_deg, RING, LOOKAHEAD):
    mesh = plsc.VectorSubcoreMesh(core_axis_name="c", subcore_axis_name="s",
                                  num_cores=2, num_subcores=NTILES)
    outs = [jax.ShapeDtypeStruct((4, NP, QD), jnp.float32)]
    if compute_deg:
        outs.append(jax.ShapeDtypeStruct((NP, DEGW), jnp.float32))
    scratch = [
        pltpu.VMEM((CPT, CHUNK), jnp.int32),
        pltpu.VMEM((CPT, CHUNK), jnp.int32),
        pltpu.VMEM((RING, CHUNK, QD), jnp.float32),
        pltpu.VMEM_SHARED((NP, QD), jnp.float32),
    ]
    if compute_deg:
        scratch += [
            pltpu.VMEM((CHUNK, DEGW), jnp.float32),
            pltpu.VMEM_SHARED((NP, DEGW), jnp.float32),
        ]
    scratch += [pltpu.SemaphoreType.DMA] * (2 * RING + 1)

    def body(*refs):
        if compute_deg:
            (h_hbm, src_hbm, dst_hbm, zeros_hbm, ones_hbm, zerosd_hbm,
             out_hbm, deg_out,
             srcbuf, dstbuf, rows, agg, ones, degacc, *sems) = refs
        else:
            (h_hbm, src_hbm, dst_hbm, zeros_hbm, out_hbm,
             srcbuf, dstbuf, rows, agg, *sems) = refs
            deg_out = ones = degacc = ones_hbm = None
        gsem = sems[:RING]
        ssem = sems[RING:2 * RING]
        dsem = sems[2 * RING]
        cid = lax.axis_index("c")
        sid = lax.axis_index("s")

        pltpu.sync_copy(src_hbm.at[cid, pl.ds(sid * CPT, CPT)], srcbuf)
        pltpu.sync_copy(dst_hbm.at[pl.ds(sid * CPT, CPT)], dstbuf)
        if compute_deg:
            pltpu.sync_copy(ones_hbm, ones)
            for z in range(RPT // CHUNK):
                pltpu.sync_copy(
                    zerosd_hbm,
                    degacc.at[pl.ds(sid * RPT + z * CHUNK, CHUNK)])

        def gather_start(j, b):
            pltpu.async_copy(h_hbm.at[srcbuf.at[j]], rows.at[b], gsem[b])

        def gather_wait(j, b):
            pltpu.make_async_copy(
                h_hbm.at[srcbuf.at[j]], rows.at[b], gsem[b]).wait()

        def scat_start(j, b):
            pltpu.async_copy(
                rows.at[b], agg.at[dstbuf.at[j]], ssem[b], add=True)

        def scat_wait(j, b):
            pltpu.make_async_copy(
                rows.at[b], agg.at[dstbuf.at[j]], ssem[b]).wait()

        for p in range(2):
            if p == 1:
                bump = jnp.full((16,), 2 * NP, jnp.int32)

                def bumploop(i, carry):
                    r = i // 8
                    c = (i % 8) * 16
                    srcbuf[r, pl.ds(c, 16)] = srcbuf[r, pl.ds(c, 16)] + bump
                    return carry

                lax.fori_loop(0, CPT * 8, bumploop, 0)

            pltpu.sync_copy(zeros_hbm, rows.at[0])
            for z in range(RPT // CHUNK):
                pltpu.sync_copy(
                    rows.at[0], agg.at[pl.ds(sid * RPT + z * CHUNK, CHUNK)])
            plsc.subcore_barrier()

            for j0 in range(LOOKAHEAD):
                gather_start(j0, j0 % RING)

            def grp(g, carry):
                for b in range(RING):
                    j = g * RING + b
                    kb = (b + LOOKAHEAD) % RING
                    k = j + LOOKAHEAD

                    @pl.when(jnp.logical_and(k < CPT, k >= RING))
                    def _():
                        scat_wait(k - RING, kb)

                    @pl.when(k < CPT)
                    def _():
                        gather_start(k, kb)

                    gather_wait(j, b)
                    scat_start(j, b)
                return carry

            lax.fori_loop(0, CPT // RING, grp, 0)
            for jt in range(CPT - RING, CPT):
                scat_wait(jt, jt % RING)

            if compute_deg and p == 0:
                @pl.when(cid == 0)
                def _():
                    def degfire(j, carry):
                        pltpu.async_copy(
                            ones, degacc.at[dstbuf.at[j]], dsem, add=True)
                        return carry

                    lax.fori_loop(0, CPT, degfire, 0)

                    def degdrain(j, carry):
                        pltpu.make_async_copy(
                            ones, degacc.at[dstbuf.at[0]], dsem).wait()
                        return carry

                    lax.fori_loop(0, CPT, degdrain, 0)

            plsc.subcore_barrier()
            pltpu.sync_copy(
                agg.at[pl.ds(sid * RPT, RPT)],
                out_hbm.at[cid + 2 * p, pl.ds(sid * RPT, RPT)])
            if compute_deg and p == 0:
                @pl.when(cid == 0)
                def _():
                    pltpu.sync_copy(degacc.at[pl.ds(sid * RPT, RPT)],
                                    deg_out.at[pl.ds(sid * RPT, RPT)])

    return pl.kernel(body, out_type=tuple(outs) if compute_deg else outs[0],
                     mesh=mesh, scratch_types=scratch,
                     compiler_params=pltpu.CompilerParams(
                         use_tc_tiling_on_sc=False))


def _tc_invdeg(degp):
    def body(d_ref, o_ref):
        deg = d_ref[...][:, 0:1]
        o_ref[...] = 1.0 / jnp.maximum(deg, 1.0)

    return pl.pallas_call(
        body, out_shape=jax.ShapeDtypeStruct((NP, 1), jnp.float32))(degp)


def _tc_layer(aggh, hh, invd, Wl, bl, Wr):
    def body(agg_ref, h_ref, invd_ref, wl_ref, bl_ref, wr_ref, y_ref, p_ref):
        g = pl.program_id(0)
        iv = invd_ref[...]
        wl = wl_ref[...]
        wr = wr_ref[...]
        y = bl_ref[...]
        for k in range(4):
            y = y + jnp.dot(agg_ref[k] * iv, wl[k * QD:(k + 1) * QD],
                            preferred_element_type=jnp.float32)
            y = y + jnp.dot(h_ref[k], wr[k * QD:(k + 1) * QD],
                            preferred_element_type=jnp.float32)
        y_ref[...] = y
        rows = g * BN_BLK + lax.broadcasted_iota(jnp.int32, (BN_BLK, 1), 0)
        ym = jnp.where(rows < N, y, 0.0)
        p = jnp.concatenate(
            [jnp.sum(ym, axis=0)[None], jnp.sum(ym * ym, axis=0)[None],
             jnp.zeros((6, D), jnp.float32)], axis=0)
        p_ref[...] = p.reshape(1, 8, D)

    return pl.pallas_call(
        body,
        grid=(GRID,),
        in_specs=[
            pl.BlockSpec((4, BN_BLK, QD), lambda g: (0, g, 0)),
            pl.BlockSpec((4, BN_BLK, QD), lambda g: (0, g, 0)),
            pl.BlockSpec((BN_BLK, 1), lambda g: (g, 0)),
            pl.BlockSpec((D, D), lambda g: (0, 0)),
            pl.BlockSpec((1, D), lambda g: (0, 0)),
            pl.BlockSpec((D, D), lambda g: (0, 0)),
        ],
        out_specs=[
            pl.BlockSpec((BN_BLK, D), lambda g: (g, 0)),
            pl.BlockSpec((1, 8, D), lambda g: (g, 0, 0)),
        ],
        out_shape=[
            jax.ShapeDtypeStruct((NP, D), jnp.float32),
            jax.ShapeDtypeStruct((GRID, 8, D), jnp.float32),
        ],
    )(aggh, hh, invd, Wl, bl, Wr)


def _tc_norm(y, p, gamma, beta):
    def body(y_ref, p_ref, g_ref, b_ref, o_ref):
        gidx = pl.program_id(0)
        ps = p_ref[...]
        mu = jnp.sum(ps[:, 0, :], axis=0) / N
        var = jnp.sum(ps[:, 1, :], axis=0) / N - mu * mu
        sc = g_ref[0] * lax.rsqrt(var + 1e-5)
        t = b_ref[0] - mu * sc
        h = jnp.maximum(y_ref[...] * sc + t, 0.0)
        rows = gidx * BN_BLK + lax.broadcasted_iota(jnp.int32, (BN_BLK, 1), 0)
        h = jnp.where(rows < N, h, 0.0)
        for k in range(4):
            o_ref[k] = h[:, k * QD:(k + 1) * QD]

    return pl.pallas_call(
        body,
        grid=(GRID,),
        in_specs=[
            pl.BlockSpec((BN_BLK, D), lambda g: (g, 0)),
            pl.BlockSpec((GRID, 8, D), lambda g: (0, 0, 0)),
            pl.BlockSpec((1, D), lambda g: (0, 0)),
            pl.BlockSpec((1, D), lambda g: (0, 0)),
        ],
        out_specs=pl.BlockSpec((4, BN_BLK, QD), lambda g: (0, g, 0)),
        out_shape=jax.ShapeDtypeStruct((4, NP, QD), jnp.float32),
    )(y, p, gamma, beta)


def _tc_head(h1, h2, h3, W1, b1, W2, b2):
    def body(h1_ref, h2_ref, h3_ref, w1_ref, b1_ref, w2_ref, b2_ref, o_ref):
        w1 = w1_ref[...]
        acc = b1_ref[...]
        for li, href in enumerate((h1_ref, h2_ref, h3_ref)):
            for k in range(4):
                acc = acc + jnp.dot(href[k], w1[4 * li + k],
                                    preferred_element_type=jnp.float32)
        u = jnp.maximum(acc, 0.0)
        v = jnp.dot(u, w2_ref[...], preferred_element_type=jnp.float32) \
            + b2_ref[...]
        m = jnp.max(v, axis=1, keepdims=True)
        lse = jnp.log(jnp.sum(jnp.exp(v - m), axis=1, keepdims=True)) + m
        o_ref[...] = v - lse

    hspec = pl.BlockSpec((4, BN_BLK, QD), lambda g: (0, g, 0))
    return pl.pallas_call(
        body,
        grid=(GRID,),
        in_specs=[
            hspec, hspec, hspec,
            pl.BlockSpec((12, QD, D), lambda g: (0, 0, 0)),
            pl.BlockSpec((1, D), lambda g: (0, 0)),
            pl.BlockSpec((D, CN), lambda g: (0, 0)),
            pl.BlockSpec((1, CN), lambda g: (0, 0)),
        ],
        out_specs=pl.BlockSpec((BN_BLK, CN), lambda g: (g, 0)),
        out_shape=jax.ShapeDtypeStruct((NP, CN), jnp.float32),
    )(h1, h2, h3, W1, b1, W2, b2)


def kernel(x, edge_index, W_l_0, b_l_0, W_r_0, bn_g_0, bn_b_0,
           W_l_1, b_l_1, W_r_1, bn_g_1, bn_b_1,
           W_l_2, b_l_2, W_r_2, bn_g_2, bn_b_2,
           lin1_W, lin1_b, lin2_W, lin2_b):
    f32 = jnp.float32
    pad = EP - E
    srcp = jnp.concatenate([edge_index[0], jnp.full((pad,), N, jnp.int32)])
    dstp = jnp.concatenate([edge_index[1], jnp.full((pad,), N, jnp.int32)])
    src2 = jnp.stack([srcp, srcp + NP]).reshape(2, EP // CHUNK, CHUNK)
    dst2 = dstp.reshape(EP // CHUNK, CHUNK)
    zeros = jnp.zeros((CHUNK, QD), f32)
    ones = jnp.ones((CHUNK, DEGW), f32)
    zerosd = jnp.zeros((CHUNK, DEGW), f32)
    hh = jnp.pad(x, ((0, NP - N), (0, 0))).reshape(NP, 4, QD).transpose(1, 0, 2)

    sc0 = _make_sc_agg(True, 5, 3)
    scn = _make_sc_agg(False, 8, 4)
    layer_params = [
        (W_l_0, b_l_0, W_r_0, bn_g_0, bn_b_0),
        (W_l_1, b_l_1, W_r_1, bn_g_1, bn_b_1),
        (W_l_2, b_l_2, W_r_2, bn_g_2, bn_b_2),
    ]
    invd = None
    feats = []
    for i, (Wl, bl, Wr, g, b) in enumerate(layer_params):
        hflat = hh.reshape(4 * NP, QD)
        if i == 0:
            aggh, degp = sc0(hflat, src2, dst2, zeros, ones, zerosd)
            invd = _tc_invdeg(degp)
        else:
            aggh = scn(hflat, src2, dst2, zeros)
        y, p = _tc_layer(aggh, hh, invd, Wl, bl.reshape(1, D), Wr)
        hh = _tc_norm(y, p, g.reshape(1, D), b.reshape(1, D))
        feats.append(hh)

    out = _tc_head(feats[0], feats[1], feats[2],
                   lin1_W.reshape(12, QD, D), lin1_b.reshape(1, D),
                   lin2_W, lin2_b.reshape(1, CN))
    return out[:N]
